# baseline (speedup 1.0000x reference)
import functools

import jax
import jax.numpy as jnp
from jax import lax
from jax.experimental import pallas as pl
from jax.experimental.pallas import tpu as pltpu
from jax.experimental.pallas import tpu_sc as plsc

N = 50000
D = 256
S = 512
NC = 2
NS = 16
NW = NC * NS
SEGW = S // NW
NPAD = N + (-N) % 16
SEARCH_ITERS = (NPAD - 1).bit_length()
CH = 112
NV = D // 16


def _lower_bound(ids_v, thresh):
    lo0 = jnp.zeros((16,), jnp.int32)
    hi0 = jnp.full((16,), NPAD, jnp.int32)

    def step(_, carry):
        lo, hi = carry
        mid = (lo + hi) // 2
        v = plsc.load_gather(ids_v, [mid])
        less = v < thresh
        lo = jnp.where(less, mid + 1, lo)
        hi = jnp.where(less, hi, mid)
        return lo, hi

    lo, _ = lax.fori_loop(0, SEARCH_ITERS, step, (lo0, hi0))
    return lo


def _sc_segment_mean(x, bidx):
    mesh = plsc.VectorSubcoreMesh(core_axis_name="c", subcore_axis_name="s")

    @functools.partial(
        pl.kernel,
        mesh=mesh,
        out_type=jax.ShapeDtypeStruct((S, D), jnp.float32),
        compiler_params=pltpu.CompilerParams(needs_layout_passes=False),
        scratch_types=[
            pltpu.VMEM((NPAD,), jnp.int32),
            pltpu.VMEM((2, CH, D), jnp.float32),
            pltpu.VMEM((SEGW, D), jnp.float32),
            pltpu.SemaphoreType.DMA((2,)),
        ],
    )
    def k(x_hbm, bidx_hbm, out_hbm, ids_v, xb, ob, sem):
        cid = lax.axis_index("c")
        sid = lax.axis_index("s")
        wid = sid * NC + cid
        base_seg = wid * SEGW

        pltpu.sync_copy(bidx_hbm, ids_v)

        tvec = base_seg + lax.iota(jnp.int32, 16)
        lo_b = _lower_bound(ids_v, tvec)
        hi_b = _lower_bound(ids_v, tvec + 1)
        counts = (hi_b - lo_b).astype(jnp.float32)
        recip = 1.0 / jnp.maximum(counts, 1.0)

        def base0_of(k_seg):
            lo8 = (lo_b[k_seg] // 8) * 8
            return pl.multiple_of(jnp.minimum(lo8, N - CH), 8)

        pltpu.async_copy(x_hbm.at[pl.ds(base0_of(0), CH)],
                         xb.at[0], sem.at[0])

        zero = jnp.zeros((16,), jnp.float32)
        for k_seg in range(SEGW):
            bk = k_seg % 2
            b_lo = lo_b[k_seg]
            b_hi = hi_b[k_seg]
            lo8 = (b_lo // 8) * 8
            nwin = (b_hi - lo8 + CH - 1) // CH

            pltpu.make_async_copy(x_hbm.at[pl.ds(0, CH)],
                                  xb.at[bk], sem.at[bk]).wait()
            if k_seg + 1 < SEGW:
                nbk = (k_seg + 1) % 2
                pltpu.async_copy(x_hbm.at[pl.ds(base0_of(k_seg + 1), CH)],
                                 xb.at[nbk], sem.at[nbk])

            def reduce_rows(acc, w_base, base_eff,
                            b_lo=b_lo, b_hi=b_hi, bk=bk):
                i_lo = jnp.maximum(b_lo, w_base) - base_eff
                i_hi = jnp.minimum(b_hi, w_base + CH) - base_eff

                n = i_hi - i_lo

                def pair(p, acc):
                    i = i_lo + 2 * p
                    acc = tuple(
                        acc[c] + xb[bk, i, pl.ds(c * 16, 16)]
                        for c in range(NV))
                    return tuple(
                        acc[c] + xb[bk, i + 1, pl.ds(c * 16, 16)]
                        for c in range(NV))

                acc = lax.fori_loop(0, n // 2, pair, acc)
                last = jnp.maximum(i_lo, i_hi - 1)
                odd = (n % 2) == 1
                return tuple(
                    acc[c] + jnp.where(odd, xb[bk, last, pl.ds(c * 16, 16)],
                                       0.0)
                    for c in range(NV))

            acc = reduce_rows((zero,) * NV, lo8,
                              pl.multiple_of(jnp.minimum(lo8, N - CH), 8))

            def window(t, acc, lo8=lo8, bk=bk):
                w_base = lo8 + t * CH
                base_eff = pl.multiple_of(jnp.minimum(w_base, N - CH), 8)
                pltpu.sync_copy(x_hbm.at[pl.ds(base_eff, CH)], xb.at[bk])
                return reduce_rows(acc, w_base, base_eff)

            acc = lax.fori_loop(1, nwin, window, acc)
            r = recip[k_seg]
            for c in range(NV):
                ob[k_seg, pl.ds(c * 16, 16)] = acc[c] * r

        pltpu.sync_copy(ob, out_hbm.at[pl.ds(base_seg, SEGW)])

    return k(x, bidx)


def kernel(x, batch):
    b = batch.astype(jnp.int32)
    bidx = jnp.concatenate([b, jnp.full((NPAD - N,), S, jnp.int32)])
    return _sc_segment_mean(x, bidx)


# speedup vs baseline: 1.0888x; 1.0888x over previous
import functools

import jax
import jax.numpy as jnp
from jax import lax
from jax.experimental import pallas as pl
from jax.experimental.pallas import tpu as pltpu
from jax.experimental.pallas import tpu_sc as plsc

N = 50000
D = 256
S = 512
NC = 2
NS = 16
NW = NC * NS
SEGW = S // NW
NPAD = N + (-N) % 16
SEARCH_ITERS = (NPAD - 1).bit_length()
CH = 112
NV = D // 16


def _lower_bound(ids_v, thresh):
    lo0 = jnp.zeros((16,), jnp.int32)
    hi0 = jnp.full((16,), NPAD, jnp.int32)

    def step(_, carry):
        lo, hi = carry
        mid = (lo + hi) // 2
        v = plsc.load_gather(ids_v, [mid])
        less = v < thresh
        lo = jnp.where(less, mid + 1, lo)
        hi = jnp.where(less, hi, mid)
        return lo, hi

    lo, _ = lax.fori_loop(0, SEARCH_ITERS, step, (lo0, hi0))
    return lo


def _sc_segment_mean(x, bidx):
    mesh = plsc.VectorSubcoreMesh(core_axis_name="c", subcore_axis_name="s")

    @functools.partial(
        pl.kernel,
        mesh=mesh,
        out_type=jax.ShapeDtypeStruct((S, D), jnp.float32),
        compiler_params=pltpu.CompilerParams(needs_layout_passes=False),
        scratch_types=[
            pltpu.VMEM((NPAD,), jnp.int32),
            pltpu.VMEM((2, CH, D), jnp.float32),
            pltpu.VMEM((SEGW, D), jnp.float32),
            pltpu.SemaphoreType.DMA((2,)),
        ],
    )
    def k(x_hbm, bidx_hbm, out_hbm, ids_v, xb, ob, sem):
        cid = lax.axis_index("c")
        sid = lax.axis_index("s")
        wid = sid * NC + cid
        base_seg = wid * SEGW

        pltpu.sync_copy(bidx_hbm, ids_v)

        tvec = base_seg + lax.iota(jnp.int32, 16)
        lo_b = _lower_bound(ids_v, tvec)
        hi_b = _lower_bound(ids_v, tvec + 1)
        counts = (hi_b - lo_b).astype(jnp.float32)
        recip = 1.0 / jnp.maximum(counts, 1.0)

        zero = jnp.zeros((16,), jnp.float32)
        for kk in range(SEGW):
            for c in range(NV):
                ob[kk, pl.ds(c * 16, 16)] = zero

        w_lo = lo_b[0]
        w_hi = hi_b[SEGW - 1]
        lo8 = (w_lo // 8) * 8
        nwin = (w_hi - lo8 + CH - 1) // CH

        def win_base(t):
            base_l = lo8 + t * CH
            base_eff = pl.multiple_of(jnp.minimum(base_l, N - CH), 8)
            return base_l, base_eff

        @pl.when(nwin > 0)
        def _prefetch0():
            pltpu.async_copy(x_hbm.at[pl.ds(win_base(0)[1], CH)],
                             xb.at[0], sem.at[0])

        def window(t, carry):
            bk = t % 2
            base_l, base_eff = win_base(t)
            pltpu.make_async_copy(x_hbm.at[pl.ds(0, CH)],
                                  xb.at[bk], sem.at[bk]).wait()

            @pl.when(t + 1 < nwin)
            def _prefetch_next():
                nbk = (t + 1) % 2
                pltpu.async_copy(x_hbm.at[pl.ds(win_base(t + 1)[1], CH)],
                                 xb.at[nbk], sem.at[nbk])

            i_lo = jnp.maximum(w_lo, base_l) - base_eff
            i_hi = jnp.minimum(w_hi, base_l + CH) - base_eff

            def row(i, carry):
                acc, k_prev = carry
                abs_r = base_eff + i
                nle = plsc.all_reduce_population_count(lo_b <= abs_r)
                k_cur = nle[0] - 1
                change = k_cur != k_prev

                @pl.when(change)
                def _flush():
                    for c in range(NV):
                        ob[k_prev, pl.ds(c * 16, 16)] = acc[c]

                acc = tuple(
                    jnp.where(change, 0.0, acc[c])
                    + xb[bk, i, pl.ds(c * 16, 16)]
                    for c in range(NV))
                return acc, k_cur

            return lax.fori_loop(i_lo, i_hi, row, carry)

        acc, k_last = lax.fori_loop(
            0, nwin, window, ((zero,) * NV, jnp.int32(0)))
        for c in range(NV):
            ob[k_last, pl.ds(c * 16, 16)] = acc[c]

        for kk in range(SEGW):
            r = recip[kk]
            for c in range(NV):
                ob[kk, pl.ds(c * 16, 16)] = ob[kk, pl.ds(c * 16, 16)] * r

        pltpu.sync_copy(ob, out_hbm.at[pl.ds(base_seg, SEGW)])

    return k(x, bidx)


def kernel(x, batch):
    b = batch.astype(jnp.int32)
    bidx = jnp.concatenate([b, jnp.full((NPAD - N,), S, jnp.int32)])
    return _sc_segment_mean(x, bidx)
